# baseline (speedup 1.0000x reference)
import functools

import jax
import jax.numpy as jnp
from jax import lax
from jax.experimental import pallas as pl
from jax.experimental.pallas import tpu as pltpu
from jax.experimental.pallas import tpu_sc as plsc

VOCAB_SIZE = 1000
DIM = 64
DIM2 = 128
B = 16384

_info = plsc.get_sparse_core_info()
_NC, _NS = _info.num_cores, _info.num_subcores
_NW = _NC * _NS
_BPW = B // _NW
_PPW = _BPW // 2
_CHUNK = 128
_NCHUNK = _PPW // _CHUNK


def _body(idx_hbm, table_hbm, out_hbm, idx_v, rows_v, sem):
    wid = lax.axis_index("s") * _NC + lax.axis_index("c")
    base = wid * _PPW
    pltpu.sync_copy(idx_hbm.at[:, pl.ds(base, _PPW)], idx_v)
    copies = [
        pltpu.async_copy(
            table_hbm.at[idx_v.at[par, pl.ds(c * _CHUNK, _CHUNK)]],
            rows_v.at[par, pl.ds(c * _CHUNK, _CHUNK)],
            sem,
        )
        for par in range(2)
        for c in range(_NCHUNK)
    ]
    for c in copies:
        c.wait()
    for par in range(2):
        pltpu.sync_copy(
            rows_v.at[par],
            out_hbm.at[pl.ds(base, _PPW), pl.ds(par * DIM, DIM)],
        )


@functools.partial(jax.jit, static_argnames=())
def kernel(indices, table):
    idx = indices.astype(jnp.int32).reshape(B // 2, 2).T
    run = pl.kernel(
        _body,
        out_type=jax.ShapeDtypeStruct((B // 2, DIM2), jnp.float32),
        mesh=plsc.VectorSubcoreMesh(core_axis_name="c", subcore_axis_name="s"),
        scratch_types=[
            pltpu.VMEM((2, _PPW), jnp.int32),
            pltpu.VMEM((2, _PPW, DIM), jnp.float32),
            pltpu.SemaphoreType.DMA,
        ],
        compiler_params=pltpu.CompilerParams(use_tc_tiling_on_sc=False),
    )
    return run(idx, table).reshape(B, DIM)


# speedup vs baseline: 1.3834x; 1.3834x over previous
import functools

import jax
import jax.numpy as jnp
from jax import lax
from jax.experimental import pallas as pl
from jax.experimental.pallas import tpu as pltpu
from jax.experimental.pallas import tpu_sc as plsc

VOCAB_SIZE = 1000
DIM = 64
DIM_PAD = 128
B = 16384

_info = plsc.get_sparse_core_info()
_NC, _NS = _info.num_cores, _info.num_subcores
_NW = _NC * _NS
_BPW = B // _NW
_CHUNK = 128
_NCHUNK = _BPW // _CHUNK


def _body(idx_hbm, table_hbm, out_hbm, idx_v, rows_v, gsem, ssem):
    wid = lax.axis_index("s") * _NC + lax.axis_index("c")
    base = wid * _BPW
    pltpu.sync_copy(idx_hbm.at[pl.ds(base, _BPW)], idx_v)
    gathers = [
        pltpu.async_copy(
            table_hbm.at[idx_v.at[pl.ds(j * _CHUNK, _CHUNK)]],
            rows_v.at[pl.ds(j * _CHUNK, _CHUNK)],
            gsem,
        )
        for j in range(_NCHUNK)
    ]
    stores = []
    for j, g in enumerate(gathers):
        g.wait()
        stores.append(
            pltpu.async_copy(
                rows_v.at[pl.ds(j * _CHUNK, _CHUNK)],
                out_hbm.at[pl.ds(base + j * _CHUNK, _CHUNK), pl.ds(0, DIM)],
                ssem,
            )
        )
    for s in stores:
        s.wait()


@functools.partial(jax.jit, static_argnames=())
def kernel(indices, table):
    idx = indices.astype(jnp.int32)
    run = pl.kernel(
        _body,
        out_type=jax.ShapeDtypeStruct((B, DIM_PAD), jnp.float32),
        mesh=plsc.VectorSubcoreMesh(core_axis_name="c", subcore_axis_name="s"),
        scratch_types=[
            pltpu.VMEM((_BPW,), jnp.int32),
            pltpu.VMEM((_BPW, DIM), jnp.float32),
            pltpu.SemaphoreType.DMA,
            pltpu.SemaphoreType.DMA,
        ],
        compiler_params=pltpu.CompilerParams(use_tc_tiling_on_sc=False),
    )
    return run(idx, table)[:, :DIM]
